# baseline (speedup 1.0000x reference)
import jax
import jax.numpy as jnp
from jax.experimental import pallas as pl

SEQ = 32
BATCH = 32
VOCAB = 100000
ROWS = SEQ * BATCH
BLOCK_ROWS = 8
GRID = ROWS // BLOCK_ROWS

_U = jnp.uint32


def _rotl(x, r):
    return jax.lax.shift_left(x, _U(r)) | jax.lax.shift_right_logical(x, _U(32 - r))


def _threefry_bits(flat_u32):
    ks0 = _U(0)
    ks1 = _U(42)
    ks2 = ks0 ^ ks1 ^ _U(0x1BD11BDA)
    ks = (ks0, ks1, ks2)
    rot = ((13, 15, 26, 6), (17, 29, 16, 24))
    x0 = jnp.full_like(flat_u32, ks0)
    x1 = flat_u32 + ks1
    for i in range(5):
        for r in rot[i % 2]:
            x0 = x0 + x1
            x1 = _rotl(x1, r)
            x1 = x1 ^ x0
        x0 = x0 + ks[(i + 1) % 3]
        x1 = x1 + ks[(i + 2) % 3] + _U(i + 1)
    return x0 ^ x1


def _gumbel_from_bits(bits):
    tiny = jnp.float32(jnp.finfo(jnp.float32).tiny)
    fb = jax.lax.shift_right_logical(bits, _U(9)) | _U(0x3F800000)
    f = jax.lax.bitcast_convert_type(fb, jnp.float32) - jnp.float32(1.0)
    u = jnp.maximum(tiny, f * (jnp.float32(1.0) - tiny) + tiny)
    return -jnp.log(-jnp.log(u))


def _row_body(x_ref, ids_ref, logp_ref):
    x = x_ref[...]
    base = pl.program_id(0) * BLOCK_ROWS
    row = jax.lax.broadcasted_iota(jnp.int32, x.shape, 0) + base
    col = jax.lax.broadcasted_iota(jnp.int32, x.shape, 1)
    flat = (row * VOCAB + col).astype(jnp.uint32)
    g = _gumbel_from_bits(_threefry_bits(flat))

    pert = x + g
    ids = jnp.argmax(pert, axis=-1).astype(jnp.int32)
    m = jnp.max(x, axis=-1)
    s = jnp.sum(jnp.exp(x - m[:, None]), axis=-1)
    lse = m + jnp.log(s)
    xat = jnp.sum(jnp.where(col == ids[:, None], x, 0.0), axis=-1)
    ids_ref[...] = ids.reshape(1, 1, BLOCK_ROWS)
    logp_ref[...] = (xat - lse).reshape(1, 1, BLOCK_ROWS)


def kernel(gen_logits):
    x2 = gen_logits.reshape(ROWS, VOCAB)

    ids3, logp3 = pl.pallas_call(
        _row_body,
        grid=(GRID,),
        in_specs=[
            pl.BlockSpec((BLOCK_ROWS, VOCAB), lambda i: (i, 0)),
        ],
        out_specs=[
            pl.BlockSpec((1, 1, BLOCK_ROWS), lambda i: (i, 0, 0)),
            pl.BlockSpec((1, 1, BLOCK_ROWS), lambda i: (i, 0, 0)),
        ],
        out_shape=[
            jax.ShapeDtypeStruct((GRID, 1, BLOCK_ROWS), jnp.int32),
            jax.ShapeDtypeStruct((GRID, 1, BLOCK_ROWS), jnp.float32),
        ],
    )(x2)

    ids = ids3.reshape(SEQ, BATCH)
    logp = logp3.reshape(SEQ, BATCH)
    generated_tensor = ids.T.astype(jnp.int64)
    return (generated_tensor, logp.T)
